# baseline (speedup 1.0000x reference)
import functools

import jax
import jax.numpy as jnp
from jax import lax
from jax.experimental import pallas as pl
from jax.experimental.pallas import tpu as pltpu
from jax.experimental.pallas import tpu_sc as plsc

A = 8192
D = 1024
B = 16
JPAD = 64
R = JPAD * B
TOTAL = 4096
OUT_LEN = 50
START_SYM = 1
STOP_SYM = 2
JBLK = 10
NC = 2
NS = 16
NW = NC * NS
RPW = R // NW
NT = 8
TW = A // NT


def _sc_gather(cu_lemma, table):
    mesh = plsc.VectorSubcoreMesh(core_axis_name="c", subcore_axis_name="s")

    @functools.partial(
        pl.kernel,
        out_type=jax.ShapeDtypeStruct((R, D), jnp.float32),
        mesh=mesh,
        scratch_types=[
            pltpu.VMEM((32 + TOTAL,), jnp.int32),
            pltpu.VMEM((RPW,), jnp.int32),
            pltpu.VMEM((RPW, D), jnp.float32),
            pltpu.SemaphoreType.DMA,
        ],
        compiler_params=pltpu.CompilerParams(needs_layout_passes=False),
    )
    def k(cl_hbm, table_hbm, out_hbm, cl_v, idx_v, rows_v, sem):
        wid = lax.axis_index("s") * NC + lax.axis_index("c")
        pltpu.sync_copy(cl_hbm, cl_v)
        io = lax.broadcasted_iota(jnp.int32, (16,), 0)
        cu16 = plsc.load_gather(cl_v, [io])
        for chunk in range(RPW // 16):
            j = wid * (RPW // 16) + chunk
            p = jnp.minimum(cu16 + j, TOTAL - 1) + 32
            tok = plsc.load_gather(cl_v, [p])
            idx_v[pl.ds(chunk * 16, 16)] = tok
        pltpu.async_copy(table_hbm.at[idx_v], rows_v, sem).wait()
        pltpu.sync_copy(rows_v, out_hbm.at[pl.ds(wid * RPW, RPW)])

    return k(cu_lemma, table)


def _tc_body(cu_ref, e_ref, wd_hbm, bd_ref, o_ref, wd16_ref, stage_ref, hs_ref, sem):
    pid = pl.program_id(0)
    m = JBLK * B

    @pl.when(pid == 0)
    def _load_wd():
        cps = [
            pltpu.make_async_copy(
                wd_hbm.at[:, pl.ds(t * TW, TW)], stage_ref.at[t % 2], sem.at[t % 2]
            )
            for t in range(NT)
        ]
        cps[0].start()
        for t in range(NT):
            if t + 1 < NT:
                cps[t + 1].start()
            cps[t].wait()
            wd16_ref[:, pl.ds(t * TW, TW)] = stage_ref[t % 2].astype(jnp.bfloat16)
        hs_ref[...] = jnp.zeros((B, 128), jnp.float32)

    r = lax.broadcasted_iota(jnp.int32, (m, 1), 0)
    jrow = pid * JBLK + r // B
    brow = r % B
    ln = jnp.zeros((m, 1), jnp.int32)
    for kk in range(B):
        lnk = cu_ref[kk + 1] - cu_ref[kk]
        ln = jnp.where(brow == kk, lnk, ln)
    valid = (jrow >= 1) & (jrow < ln)
    e = e_ref[...].reshape(m, D) * valid.astype(jnp.float32)
    x = jnp.dot(e.astype(jnp.bfloat16), wd16_ref[...],
                preferred_element_type=jnp.float32)
    x = x + bd_ref[...]
    s = jax.nn.sigmoid(x)
    ex = jnp.exp(s)
    denom = jnp.sum(ex, axis=-1, keepdims=True)
    y = ex / denom
    xmax = jnp.max(x, axis=-1, keepdims=True)
    lane = lax.broadcasted_iota(jnp.int32, (m, A), 1)
    sym = jnp.min(jnp.where(x == xmax, lane, A), axis=-1, keepdims=True)
    hit = ((sym == STOP_SYM) & (jrow >= 1)).astype(jnp.float32)
    r_i = lax.broadcasted_iota(jnp.int32, (m, m), 0)
    r_k = lax.broadcasted_iota(jnp.int32, (m, m), 1)
    same_b = (r_i % B) == (r_k % B)
    tril = (same_b & (r_k < r_i)).astype(jnp.bfloat16)
    prefix = jnp.dot(tril, hit.astype(jnp.bfloat16),
                     preferred_element_type=jnp.float32)
    sel_i = lax.broadcasted_iota(jnp.int32, (m, B), 0)
    sel_k = lax.broadcasted_iota(jnp.int32, (m, B), 1)
    sel = ((sel_i % B) == sel_k).astype(jnp.bfloat16)
    prior = jnp.dot(sel, hs_ref[:, 0:1].astype(jnp.bfloat16),
                    preferred_element_type=jnp.float32)
    keep = (prior + prefix) == 0.0
    stoprow = (lane == STOP_SYM).astype(jnp.float32)
    body = jnp.where(keep, y, stoprow)
    st_i = lax.broadcasted_iota(jnp.int32, (B, m), 0)
    st_k = lax.broadcasted_iota(jnp.int32, (B, m), 1)
    selT = (st_i == (st_k % B)).astype(jnp.bfloat16)
    upd = jnp.dot(selT, hit.astype(jnp.bfloat16),
                  preferred_element_type=jnp.float32)
    hs_ref[:, 0:1] = hs_ref[:, 0:1] + upd
    o_ref[...] = body.reshape(JBLK, B, A)

    @pl.when(pid == 0)
    def _start_row():
        srow = (lane[0:1, :] == START_SYM).astype(jnp.float32)
        o_ref[0:1, :, :] = jnp.broadcast_to(srow[:, None, :], (1, B, A))


def _tc_decode(cu_pad, e_all, wd, bd_row):
    out = pl.pallas_call(
        _tc_body,
        grid=(OUT_LEN // JBLK,),
        in_specs=[
            pl.BlockSpec(memory_space=pltpu.SMEM),
            pl.BlockSpec((JBLK, B, D), lambda i: (i, 0, 0)),
            pl.BlockSpec(memory_space=pl.ANY),
            pl.BlockSpec((1, A), lambda i: (0, 0)),
        ],
        out_specs=pl.BlockSpec((JBLK, B, A), lambda i: (i, 0, 0)),
        out_shape=jax.ShapeDtypeStruct((OUT_LEN, B, A), jnp.float32),
        scratch_shapes=[
            pltpu.VMEM((D, A), jnp.bfloat16),
            pltpu.VMEM((2, D, TW), jnp.float32),
            pltpu.VMEM((B, 128), jnp.float32),
            pltpu.SemaphoreType.DMA((2,)),
        ],
        compiler_params=pltpu.CompilerParams(
            dimension_semantics=("arbitrary",),
            vmem_limit_bytes=128 * 1024 * 1024,
        ),
    )(cu_pad, e_all, wd, bd_row)
    return jnp.transpose(out, (1, 0, 2))


def kernel(lemma_flat, cu_seqlens, table, Wd, bd, W1, b1, W2, b2, W3, b3):
    cu_pad = jnp.pad(cu_seqlens, (0, 32 - cu_seqlens.shape[0]))
    cu_lemma = jnp.concatenate([cu_pad, lemma_flat])
    e_rows = _sc_gather(cu_lemma, table)
    e_all = e_rows.reshape(JPAD, B, D)
    bd_row = bd.reshape(1, A)
    return _tc_decode(cu_pad, e_all, Wd, bd_row)
